# baseline (speedup 1.0000x reference)
import functools

import jax
import jax.numpy as jnp
from jax import lax
from jax.experimental import pallas as pl
from jax.experimental.pallas import tpu as pltpu
from jax.experimental.pallas import tpu_sc as plsc

_NE = 1024
_DIM = 64
_PIX = 1024


def _vq_block(z_ref, e_ref, idx_ref, loss_ref):
    z_pm = z_ref[...]
    e = e_ref[...]

    z = jnp.transpose(z_pm, (1, 0))

    z2 = (2.0 * z).astype(jnp.bfloat16)
    eb = e.astype(jnp.bfloat16)
    dot2 = jnp.dot(eb, z2, preferred_element_type=jnp.float32)

    z_norm = jnp.sum(z * z, axis=0, keepdims=True)
    e_norm = jnp.sum(e * e, axis=1, keepdims=True)
    dist = (z_norm - dot2) + e_norm

    min_val = jnp.min(dist, axis=0, keepdims=True)
    iota = jax.lax.broadcasted_iota(jnp.int32, (_NE, _PIX), 0)
    idx = jnp.min(jnp.where(dist == min_val, iota, _NE),
                  axis=0, keepdims=True)

    idx_ref[0] = idx.reshape(8, 128)
    loss_ref[...] = jnp.sum(min_val).reshape(1, 1, 1)


def _sc_gather(table128, idx_flat, n_rows):
    info = plsc.get_sparse_core_info()
    nw = info.num_cores * info.num_subcores
    b_per_w = n_rows // nw
    n_chunks = 2
    chunk = b_per_w // n_chunks
    mesh = plsc.VectorSubcoreMesh(core_axis_name="c", subcore_axis_name="s")

    @functools.partial(
        pl.kernel, mesh=mesh,
        out_type=jax.ShapeDtypeStruct((n_rows, 128), jnp.float32),
        scratch_types=[
            pltpu.VMEM((chunk,), jnp.int32),
            pltpu.VMEM((chunk, 128), jnp.float32),
            pltpu.SemaphoreType.DMA,
        ],
    )
    def gather_kernel(table_hbm, idx_hbm, out_hbm, idx_v, rows_v, sem):
        wid = (lax.axis_index("s") * info.num_cores + lax.axis_index("c"))
        for j in range(n_chunks):
            base = wid * b_per_w + j * chunk
            pltpu.sync_copy(idx_hbm.at[pl.ds(base, chunk)], idx_v)
            pltpu.async_copy(table_hbm.at[idx_v], rows_v, sem).wait()
            pltpu.sync_copy(rows_v, out_hbm.at[pl.ds(base, chunk)])

    return gather_kernel(table128, idx_flat)


def kernel(z_e, embedding):
    B, C, D, H, W = z_e.shape
    npix = B * D * H * W
    n_blocks = npix // _PIX
    zf = jnp.transpose(z_e, (0, 2, 3, 4, 1)).reshape(npix, C)

    idx, loss_parts = pl.pallas_call(
        _vq_block,
        grid=(n_blocks,),
        in_specs=[
            pl.BlockSpec((_PIX, C), lambda i: (i, 0)),
            pl.BlockSpec((_NE, _DIM), lambda i: (0, 0)),
        ],
        out_specs=[
            pl.BlockSpec((1, 8, 128), lambda i: (i, 0, 0)),
            pl.BlockSpec((1, 1, 1), lambda i: (i, 0, 0)),
        ],
        out_shape=[
            jax.ShapeDtypeStruct((n_blocks, 8, 128), jnp.int32),
            jax.ShapeDtypeStruct((n_blocks, 1, 1), jnp.float32),
        ],
    )(zf, embedding)

    idx_flat = idx.reshape(npix)
    table128 = jnp.pad(embedding, ((0, 0), (0, 128 - C)))
    q = _sc_gather(table128, idx_flat, npix)[:, :C]

    loss = jnp.sum(loss_parts) / (npix * C)
    quantized_st = jnp.transpose(q.reshape(B, D, H, W, C), (0, 4, 1, 2, 3))
    encoding_indices = idx.reshape(B, D, H, W)
    return quantized_st, loss, loss, encoding_indices


# speedup vs baseline: 1.4626x; 1.4626x over previous
import jax
import jax.numpy as jnp
from jax.experimental import pallas as pl
from jax.experimental.pallas import tpu as pltpu

_NE = 1024
_DIM = 64
_PIX = 1024


def _vq_block(z_ref, e_ref, et_hi_ref, q_ref, idx_ref, loss_ref):
    z_pm = z_ref[...]
    e = e_ref[...]
    et_hi = et_hi_ref[...]

    z = jnp.transpose(z_pm, (1, 0))

    z2 = (2.0 * z).astype(jnp.bfloat16)
    eb = e.astype(jnp.bfloat16)
    dot2 = jnp.dot(eb, z2, preferred_element_type=jnp.float32)

    z_norm = jnp.sum(z * z, axis=0, keepdims=True)
    e_norm = jnp.sum(e * e, axis=1, keepdims=True)
    dist = (z_norm - dot2) + e_norm

    min_val = jnp.min(dist, axis=0, keepdims=True)
    iota = jax.lax.broadcasted_iota(jnp.int32, (_NE, _PIX), 0)
    idx = jnp.min(jnp.where(dist == min_val, iota, _NE),
                  axis=0, keepdims=True)

    onehot = (iota == idx).astype(jnp.bfloat16)
    q = jnp.dot(et_hi, onehot, preferred_element_type=jnp.float32)
    q_pm = jnp.transpose(q, (1, 0))

    q_ref[...] = z_pm + (q_pm - z_pm)
    idx_ref[0] = idx.reshape(8, 128)
    loss_ref[...] = jnp.sum(min_val).reshape(1, 1, 1)


def kernel(z_e, embedding):
    B, C, D, H, W = z_e.shape
    npix = B * D * H * W
    n_blocks = npix // _PIX
    zf = jnp.transpose(z_e, (0, 2, 3, 4, 1)).reshape(npix, C)
    et_hi = embedding.T.astype(jnp.bfloat16)

    q, idx, loss_parts = pl.pallas_call(
        _vq_block,
        grid=(n_blocks,),
        in_specs=[
            pl.BlockSpec((_PIX, C), lambda i: (i, 0)),
            pl.BlockSpec((_NE, _DIM), lambda i: (0, 0)),
            pl.BlockSpec((_DIM, _NE), lambda i: (0, 0)),
        ],
        out_specs=[
            pl.BlockSpec((_PIX, C), lambda i: (i, 0)),
            pl.BlockSpec((1, 8, 128), lambda i: (i, 0, 0)),
            pl.BlockSpec((1, 1, 1), lambda i: (i, 0, 0)),
        ],
        out_shape=[
            jax.ShapeDtypeStruct((npix, C), jnp.float32),
            jax.ShapeDtypeStruct((n_blocks, 8, 128), jnp.int32),
            jax.ShapeDtypeStruct((n_blocks, 1, 1), jnp.float32),
        ],
    )(zf, embedding, et_hi)

    loss = jnp.sum(loss_parts) / (npix * C)
    quantized_st = jnp.transpose(q.reshape(B, D, H, W, C), (0, 4, 1, 2, 3))
    encoding_indices = idx.reshape(B, D, H, W)
    return quantized_st, loss, loss, encoding_indices
